# baseline (speedup 1.0000x reference)
import functools

import jax
import jax.numpy as jnp
from jax import lax
from jax.experimental import pallas as pl
from jax.experimental.pallas import tpu as pltpu
from jax.experimental.pallas import tpu_sc as plsc

N = 10000
NP = 10240
E = 320000
F_IN = 128
H = 128
C = 64

NC = 2
NS = 16
NW = NC * NS
L = 16
FPW = H // NW
EPW = 10240
EP = NW * EPW
CE = 2048
NCHK = EP // CE

_MESH = dict(core_axis_name="c", subcore_axis_name="s",
             num_cores=NC, num_subcores=NS)
_SC_PARAMS = pltpu.CompilerParams(needs_layout_passes=False)


def _sc_degree(dst2, ew2):
    @functools.partial(
        pl.kernel,
        out_type=jax.ShapeDtypeStruct((NW, NP), jnp.float32),
        mesh=plsc.VectorSubcoreMesh(**_MESH),
        compiler_params=_SC_PARAMS,
        scratch_types=[
            pltpu.VMEM((EPW,), jnp.int32),
            pltpu.VMEM((EPW,), jnp.float32),
            pltpu.VMEM((NP,), jnp.float32),
        ],
    )
    def k(dst_hbm, ew_hbm, out_hbm, dst_v, ew_v, acc):
        cid = lax.axis_index("c")
        sid = lax.axis_index("s")
        wid = sid * NC + cid
        pltpu.sync_copy(dst_hbm.at[wid], dst_v)
        pltpu.sync_copy(ew_hbm.at[wid], ew_v)

        zeros = jnp.zeros((L,), jnp.float32)

        def zbody(i, _):
            acc[pl.ds(i * L, L)] = zeros
            return 0

        lax.fori_loop(0, NP // L, zbody, 0)

        def ebody(i, _):
            for u in range(4):
                idx = dst_v[pl.ds(i * 4 * L + u * L, L)]
                w = ew_v[pl.ds(i * 4 * L + u * L, L)]
                plsc.addupdate_scatter(acc, [idx], w)
            return 0

        lax.fori_loop(0, EPW // (4 * L), ebody, 0)
        pltpu.sync_copy(acc, out_hbm.at[wid])

    return k(dst2, ew2)


def _sc_aggregate_t(gt, srcC, dstC, ewC):
    @functools.partial(
        pl.kernel,
        out_type=jax.ShapeDtypeStruct((NW, FPW * NP), jnp.float32),
        mesh=plsc.VectorSubcoreMesh(**_MESH),
        compiler_params=_SC_PARAMS,
        scratch_types=[
            pltpu.VMEM((FPW * NP,), jnp.float32),
            pltpu.VMEM((FPW * NP,), jnp.float32),
            pltpu.VMEM((CE,), jnp.int32),
            pltpu.VMEM((CE,), jnp.int32),
            pltpu.VMEM((CE,), jnp.int32),
            pltpu.VMEM((CE,), jnp.int32),
            pltpu.VMEM((CE,), jnp.float32),
            pltpu.VMEM((CE,), jnp.float32),
            pltpu.SemaphoreType.DMA,
            pltpu.SemaphoreType.DMA,
        ],
    )
    def k(g_hbm, src_hbm, dst_hbm, ew_hbm, out_hbm,
          gl, acc, s0, s1, d0, d1, w0, w1, sem0, sem1):
        cid = lax.axis_index("c")
        sid = lax.axis_index("s")
        wid = sid * NC + cid
        sv = (s0, s1)
        dv = (d0, d1)
        wv = (w0, w1)
        sem = (sem0, sem1)

        pltpu.sync_copy(g_hbm.at[wid], gl)

        zeros = jnp.zeros((L,), jnp.float32)

        def zbody(i, _):
            for u in range(4):
                acc[pl.ds(i * 4 * L + u * L, L)] = zeros
            return 0

        lax.fori_loop(0, FPW * NP // (4 * L), zbody, 0)

        def stage(ch, p):
            pltpu.async_copy(src_hbm.at[ch], sv[p], sem[p])
            pltpu.async_copy(dst_hbm.at[ch], dv[p], sem[p])
            pltpu.async_copy(ew_hbm.at[ch], wv[p], sem[p])

        def stage_wait(ch, p):
            pltpu.make_async_copy(src_hbm.at[ch], sv[p], sem[p]).wait()
            pltpu.make_async_copy(dst_hbm.at[ch], dv[p], sem[p]).wait()
            pltpu.make_async_copy(ew_hbm.at[ch], wv[p], sem[p]).wait()

        foff = tuple(
            jnp.full((L,), f * NP, jnp.int32) for f in range(1, FPW))

        def compute(p):
            def group(j, _):
                sl = pl.ds(j * L, L)
                s16 = sv[p][sl]
                d16 = dv[p][sl]
                w16 = wv[p][sl]
                g16 = plsc.load_gather(gl, [s16])
                plsc.addupdate_scatter(acc, [d16], g16 * w16)
                for f in range(1, FPW):
                    g16 = plsc.load_gather(gl, [s16 + foff[f - 1]])
                    plsc.addupdate_scatter(acc, [d16 + foff[f - 1]],
                                           g16 * w16)
                return 0

            lax.fori_loop(0, CE // L, group, 0)

        stage(0, 0)

        def pair(i, _):
            for p in range(2):
                ch = i * 2 + p
                q = 1 - p
                stage_wait(ch, p)

                @pl.when(ch + 1 < NCHK)
                def _():
                    stage(ch + 1, q)

                compute(p)
            return 0

        lax.fori_loop(0, NCHK // 2, pair, 0)
        pltpu.sync_copy(acc, out_hbm.at[wid])

    return k(gt, srcC, dstC, ewC)


_R = 1024
_G = NP // _R


def _tc_prep_t(degp, xT, W1T):
    def body(deg_ref, x_ref, w_ref, g_ref):
        deg = jnp.sum(deg_ref[...], axis=0) + 1.0
        di = lax.rsqrt(deg)[None, :]
        g = jnp.dot(w_ref[...], x_ref[...],
                    preferred_element_type=jnp.float32)
        g_ref[...] = di * g

    return pl.pallas_call(
        body,
        grid=(_G,),
        in_specs=[
            pl.BlockSpec((NW, _R), lambda i: (0, i)),
            pl.BlockSpec((F_IN, _R), lambda i: (0, i)),
            pl.BlockSpec((H, F_IN), lambda i: (0, 0)),
        ],
        out_specs=pl.BlockSpec((H, _R), lambda i: (0, i)),
        out_shape=jax.ShapeDtypeStruct((H, NP), jnp.float32),
    )(degp, xT, W1T)


def _tc_mid_t(acc1t, degp, g1t, b1c):
    def body(acc_ref, deg_ref, g_ref, b_ref, q2_ref):
        deg = jnp.sum(deg_ref[...], axis=0) + 1.0
        di = lax.rsqrt(deg)[None, :]
        a = acc_ref[...] + g_ref[...]
        out1 = jnp.maximum(di * a + b_ref[...], 0.0)
        q2_ref[...] = di * out1

    return pl.pallas_call(
        body,
        grid=(_G,),
        in_specs=[
            pl.BlockSpec((H, _R), lambda i: (0, i)),
            pl.BlockSpec((NW, _R), lambda i: (0, i)),
            pl.BlockSpec((H, _R), lambda i: (0, i)),
            pl.BlockSpec((H, 1), lambda i: (0, 0)),
        ],
        out_specs=pl.BlockSpec((H, _R), lambda i: (0, i)),
        out_shape=jax.ShapeDtypeStruct((H, NP), jnp.float32),
    )(acc1t, degp, g1t, b1c)


def _tc_final_t(acc2t, degp, q2t, W2T, b2c):
    def body(acc_ref, deg_ref, q_ref, w_ref, b_ref, o_ref):
        deg = jnp.sum(deg_ref[...], axis=0) + 1.0
        di = lax.rsqrt(deg)[None, :]
        a = di * (acc_ref[...] + q_ref[...])
        o = jnp.dot(w_ref[...], a,
                    preferred_element_type=jnp.float32) + b_ref[...]
        nrm = jnp.sqrt(jnp.sum(o * o, axis=0, keepdims=True))
        o_ref[...] = (o / jnp.maximum(nrm, 1e-12)).T

    return pl.pallas_call(
        body,
        grid=(_G,),
        in_specs=[
            pl.BlockSpec((H, _R), lambda i: (0, i)),
            pl.BlockSpec((NW, _R), lambda i: (0, i)),
            pl.BlockSpec((H, _R), lambda i: (0, i)),
            pl.BlockSpec((C, H), lambda i: (0, 0)),
            pl.BlockSpec((C, 1), lambda i: (0, 0)),
        ],
        out_specs=pl.BlockSpec((_R, C), lambda i: (i, 0)),
        out_shape=jax.ShapeDtypeStruct((NP, C), jnp.float32),
    )(acc2t, degp, q2t, W2T, b2c)


def kernel(x, edge_index, edge_weight, W1, b1, W2, b2):
    src_p = jnp.pad(edge_index[0], (0, EP - E))
    dst_p = jnp.pad(edge_index[1], (0, EP - E))
    ew_p = jnp.pad(edge_weight, (0, EP - E))
    srcC = src_p.reshape(NCHK, CE)
    dstC = dst_p.reshape(NCHK, CE)
    ewC = ew_p.reshape(NCHK, CE)
    dst2 = dst_p.reshape(NW, EPW)
    ew2 = ew_p.reshape(NW, EPW)
    xT = jnp.pad(x, ((0, NP - N), (0, 0))).T

    degp = _sc_degree(dst2, ew2)
    g1t = _tc_prep_t(degp, xT, W1.T)
    acc1 = _sc_aggregate_t(g1t.reshape(NW, FPW * NP),
                           srcC, dstC, ewC).reshape(H, NP)
    q2t = _tc_mid_t(acc1, degp, g1t, b1.reshape(H, 1))
    acc2 = _sc_aggregate_t(q2t.reshape(NW, FPW * NP),
                           srcC, dstC, ewC).reshape(H, NP)
    out = _tc_final_t(acc2, degp, q2t, W2.T, b2.reshape(C, 1))
    return out[:N]


# speedup vs baseline: 2.2623x; 2.2623x over previous
import functools

import jax
import jax.numpy as jnp
from jax import lax
from jax.experimental import pallas as pl
from jax.experimental.pallas import tpu as pltpu
from jax.experimental.pallas import tpu_sc as plsc

N = 10000
NP = 10240
E = 320000
F_IN = 128
H = 128
C = 64

NC = 2
NS = 16
NW = NC * NS
L = 16
EPW = E // NW
B = 80
NBLK = EPW // B
RPT = NP // NS
CHB = 25
CHE = CHB * B
NCH = NBLK // CHB

_MESH = dict(core_axis_name="c", subcore_axis_name="s",
             num_cores=NC, num_subcores=NS)
_SC_PARAMS = pltpu.CompilerParams(needs_layout_passes=False)

_GDN = lax.GatherDimensionNumbers(
    offset_dims=(), collapsed_slice_dims=(0,), start_index_map=(0,))


def _lane_bcast(v16, j):
    return lax.gather(v16, jnp.full((L, 1), j, jnp.int32), _GDN, (1,),
                      mode=lax.GatherScatterMode.PROMISE_IN_BOUNDS)


def _sc_degree(dst2, ew2):
    @functools.partial(
        pl.kernel,
        out_type=jax.ShapeDtypeStruct((NW, NP), jnp.float32),
        mesh=plsc.VectorSubcoreMesh(**_MESH),
        compiler_params=_SC_PARAMS,
        scratch_types=[
            pltpu.VMEM((EPW,), jnp.int32),
            pltpu.VMEM((EPW,), jnp.float32),
            pltpu.VMEM((NP,), jnp.float32),
        ],
    )
    def k(dst_hbm, ew_hbm, out_hbm, dst_v, ew_v, acc):
        cid = lax.axis_index("c")
        sid = lax.axis_index("s")
        wid = sid * NC + cid
        pltpu.sync_copy(dst_hbm.at[wid], dst_v)
        pltpu.sync_copy(ew_hbm.at[wid], ew_v)

        zeros = jnp.zeros((L,), jnp.float32)

        def zbody(i, _):
            acc[pl.ds(i * L, L)] = zeros
            return 0

        lax.fori_loop(0, NP // L, zbody, 0)

        def ebody(i, _):
            idx = dst_v[pl.ds(i * L, L)]
            w = ew_v[pl.ds(i * L, L)]
            plsc.addupdate_scatter(acc, [idx], w)
            return 0

        lax.fori_loop(0, EPW // L, ebody, 0)
        pltpu.sync_copy(acc, out_hbm.at[wid])

    return k(dst2, ew2)


def _make_sc_aggregate(D):
    @functools.partial(
        pl.kernel,
        out_type=jax.ShapeDtypeStruct((NC, NP, D), jnp.float32),
        mesh=plsc.VectorSubcoreMesh(**_MESH),
        compiler_params=_SC_PARAMS,
        scratch_types=[
            pltpu.VMEM((CHE,), jnp.int32),
            pltpu.VMEM((CHE,), jnp.int32),
            pltpu.VMEM((CHE,), jnp.float32),
            pltpu.VMEM((B,), jnp.int32),
            pltpu.VMEM((B,), jnp.int32),
            pltpu.VMEM((B, D), jnp.float32),
            pltpu.VMEM_SHARED((NP, D), jnp.float32),
            pltpu.SemaphoreType.DMA,
        ],
    )
    def k(g_hbm, src_hbm, dst_hbm, ew_hbm, out_hbm,
          src_all, dst_all, ew_all, src_v, dst_v, rows, acc_sh, sem):
        cid = lax.axis_index("c")
        sid = lax.axis_index("s")
        wid = sid * NC + cid

        zeros = jnp.zeros((L,), jnp.float32)

        def zbody(i, _):
            for kk in range(D // L):
                rows[i, pl.ds(kk * L, L)] = zeros
            return 0

        lax.fori_loop(0, B, zbody, 0)
        rbase = sid * RPT
        for cchunk in range(RPT // B):
            pltpu.sync_copy(rows, acc_sh.at[pl.ds(rbase + cchunk * B, B)])
        plsc.subcore_barrier()

        def chunk(ch, _):
            pltpu.sync_copy(src_hbm.at[wid, ch], src_all)
            pltpu.sync_copy(dst_hbm.at[wid, ch], dst_all)
            pltpu.sync_copy(ew_hbm.at[wid, ch], ew_all)

            def block(b, _):
                off = b * B
                for g in range(B // L):
                    src_v[pl.ds(g * L, L)] = src_all[pl.ds(off + g * L, L)]
                    dst_v[pl.ds(g * L, L)] = dst_all[pl.ds(off + g * L, L)]
                pltpu.async_copy(g_hbm.at[src_v], rows, sem).wait()

                def sgroup(g, _):
                    w16 = ew_all[pl.ds(off + g * L, L)]
                    for j in range(L):
                        e = g * L + j
                        wspl = _lane_bcast(w16, j)
                        for kk in range(D // L):
                            rows[e, pl.ds(kk * L, L)] = (
                                rows[e, pl.ds(kk * L, L)] * wspl)
                    return 0

                lax.fori_loop(0, B // L, sgroup, 0)
                pltpu.sync_copy(rows, acc_sh.at[dst_v], add=True)
                return 0

            lax.fori_loop(0, CHB, block, 0)
            return 0

        lax.fori_loop(0, NCH, chunk, 0)
        plsc.subcore_barrier()
        pltpu.sync_copy(acc_sh.at[pl.ds(rbase, RPT)],
                        out_hbm.at[cid, pl.ds(rbase, RPT)])

    return k


_sc_aggregate_h = _make_sc_aggregate(H)


_R = 1024
_G = NP // _R


def _tc_prep(degp, x_pad, W1):
    def body(deg_ref, x_ref, w_ref, g_ref):
        deg = jnp.sum(deg_ref[...], axis=0) + 1.0
        di = lax.rsqrt(deg)[:, None]
        h = jnp.dot(x_ref[...], w_ref[...],
                    preferred_element_type=jnp.float32)
        g_ref[...] = di * h

    return pl.pallas_call(
        body,
        grid=(_G,),
        in_specs=[
            pl.BlockSpec((NW, _R), lambda i: (0, i)),
            pl.BlockSpec((_R, F_IN), lambda i: (i, 0)),
            pl.BlockSpec((F_IN, H), lambda i: (0, 0)),
        ],
        out_specs=pl.BlockSpec((_R, H), lambda i: (i, 0)),
        out_shape=jax.ShapeDtypeStruct((NP, H), jnp.float32),
    )(degp, x_pad, W1)


def _tc_mid(acc1p, degp, g1, b1r):
    def body(acc_ref, deg_ref, g_ref, b_ref, q2_ref):
        deg = jnp.sum(deg_ref[...], axis=0) + 1.0
        di = lax.rsqrt(deg)[:, None]
        a = acc_ref[0] + acc_ref[1] + g_ref[...]
        out1 = jnp.maximum(di * a + b_ref[...], 0.0)
        q2_ref[...] = di * out1

    return pl.pallas_call(
        body,
        grid=(_G,),
        in_specs=[
            pl.BlockSpec((NC, _R, H), lambda i: (0, i, 0)),
            pl.BlockSpec((NW, _R), lambda i: (0, i)),
            pl.BlockSpec((_R, H), lambda i: (i, 0)),
            pl.BlockSpec((1, H), lambda i: (0, 0)),
        ],
        out_specs=pl.BlockSpec((_R, H), lambda i: (i, 0)),
        out_shape=jax.ShapeDtypeStruct((NP, H), jnp.float32),
    )(acc1p, degp, g1, b1r)


def _tc_final(acc2p, degp, q2, W2, b2r):
    def body(acc_ref, deg_ref, q_ref, w_ref, b_ref, o_ref):
        deg = jnp.sum(deg_ref[...], axis=0) + 1.0
        di = lax.rsqrt(deg)[:, None]
        a = di * (acc_ref[0] + acc_ref[1] + q_ref[...])
        o = jnp.dot(a, w_ref[...],
                    preferred_element_type=jnp.float32) + b_ref[...]
        nrm = jnp.sqrt(jnp.sum(o * o, axis=1, keepdims=True))
        o_ref[...] = o / jnp.maximum(nrm, 1e-12)

    return pl.pallas_call(
        body,
        grid=(_G,),
        in_specs=[
            pl.BlockSpec((NC, _R, H), lambda i: (0, i, 0)),
            pl.BlockSpec((NW, _R), lambda i: (0, i)),
            pl.BlockSpec((_R, H), lambda i: (i, 0)),
            pl.BlockSpec((H, C), lambda i: (0, 0)),
            pl.BlockSpec((1, C), lambda i: (0, 0)),
        ],
        out_specs=pl.BlockSpec((_R, C), lambda i: (i, 0)),
        out_shape=jax.ShapeDtypeStruct((NP, C), jnp.float32),
    )(acc2p, degp, q2, W2, b2r)


def kernel(x, edge_index, edge_weight, W1, b1, W2, b2):
    src3 = edge_index[0].reshape(NW, NCH, CHE)
    dst3 = edge_index[1].reshape(NW, NCH, CHE)
    ew3 = edge_weight.reshape(NW, NCH, CHE)
    dst2 = edge_index[1].reshape(NW, EPW)
    ew2 = edge_weight.reshape(NW, EPW)
    x_pad = jnp.pad(x, ((0, NP - N), (0, 0)))

    degp = _sc_degree(dst2, ew2)
    g1 = _tc_prep(degp, x_pad, W1)
    acc1p = _sc_aggregate_h(g1, src3, dst3, ew3)
    q2 = _tc_mid(acc1p, degp, g1, b1.reshape(1, H))
    acc2p = _sc_aggregate_h(q2, src3, dst3, ew3)
    out = _tc_final(acc2p, degp, q2, W2, b2.reshape(1, C))
    return out[:N]
